# baseline (speedup 1.0000x reference)
import functools

import jax
import jax.numpy as jnp
from jax import lax
from jax.experimental import pallas as pl
from jax.experimental.pallas import tpu as pltpu
from jax.experimental.pallas import tpu_sc as plsc

_RATIO = 0.7
_LMBDA = 0.5
_B = 16384
_C = 1000
_K = int(_B * _RATIO)

_NW = 32
_RPW = _B // _NW
_G = 16
_NG = _RPW // _G


def _sc_phase1_body(yh_hbm, y_hbm, nll_hbm, tl_hbm, byh, byy, snll, stl):
    wid = lax.axis_index("s") * 2 + lax.axis_index("c")
    base_row = wid * _RPW
    lane = lax.iota(jnp.int32, 16)
    lane_base = lane * _C

    def group_body(g, _):
        row0 = base_row + g * _G
        pltpu.sync_copy(yh_hbm.at[pl.ds(row0 * _C, _G * _C)], byh)
        pltpu.sync_copy(y_hbm.at[pl.ds(row0 * _C, _G * _C)], byy)

        def col_body(j, carry):
            col, dot, ym, yhm, cb = carry
            idx = lane_base + col
            yh = plsc.load_gather(byh, [idx])
            yy = plsc.load_gather(byy, [idx])
            better = (yy > ym) | ((yy == ym) & (col < cb))
            ym = jnp.where(better, yy, ym)
            yhm = jnp.where(better, yh, yhm)
            cb = jnp.where(better, col, cb)
            dot = dot + yh * yy
            col = col + 1
            col = jnp.where(col == _C, 0, col)
            return col, dot, ym, yhm, cb

        col0 = lane
        dot0 = jnp.zeros((16,), jnp.float32)
        ym0 = jnp.full((16,), -1.0, jnp.float32)
        yhm0 = jnp.zeros((16,), jnp.float32)
        cb0 = jnp.full((16,), _C, jnp.int32)
        _, dot, _, yhm, _ = lax.fori_loop(
            0, _C, col_body, (col0, dot0, ym0, yhm0, cb0))
        nll = -yhm
        snll[pl.ds(g * _G, _G)] = nll
        stl[pl.ds(g * _G, _G)] = nll + _LMBDA * (1.0 - dot)
        return 0

    lax.fori_loop(0, _NG, group_body, 0)
    pltpu.sync_copy(snll, nll_hbm.at[pl.ds(base_row, _RPW)])
    pltpu.sync_copy(stl, tl_hbm.at[pl.ds(base_row, _RPW)])


_sc_phase1 = functools.partial(
    pl.kernel,
    out_type=[
        jax.ShapeDtypeStruct((_B,), jnp.float32),
        jax.ShapeDtypeStruct((_B,), jnp.float32),
    ],
    mesh=plsc.VectorSubcoreMesh(core_axis_name="c", subcore_axis_name="s"),
    scratch_types=[
        pltpu.VMEM((_G * _C,), jnp.float32),
        pltpu.VMEM((_G * _C,), jnp.float32),
        pltpu.VMEM((_RPW,), jnp.float32),
        pltpu.VMEM((_RPW,), jnp.float32),
    ],
    compiler_params=pltpu.CompilerParams(needs_layout_passes=False),
)(_sc_phase1_body)


def _phase2_body(nll_ref, tl_ref, out_ref):
    nll = nll_ref[...]
    tl = tl_ref[...]
    i32 = lax.bitcast_convert_type(tl, jnp.int32)
    keyi = jnp.where(i32 < 0, jnp.bitwise_not(i32),
                     jnp.bitwise_or(i32, jnp.int32(-(2**31))))
    u = lax.bitcast_convert_type(keyi, jnp.uint32)
    t = jnp.uint32(0)
    for b in range(31, -1, -1):
        cand = t | jnp.uint32(1 << b)
        cnt = jnp.sum((u >= cand).astype(jnp.int32))
        t = jnp.where(cnt >= _K, cand, t)
    gt = u > t
    eq = u == t
    cnt_gt = jnp.sum(gt.astype(jnp.int32))
    sum_gt = jnp.sum(jnp.where(gt, nll, 0.0))
    cnt_eq = jnp.sum(eq.astype(jnp.int32))
    sum_eq = jnp.sum(jnp.where(eq, nll, 0.0))
    rem = (_K - cnt_gt).astype(jnp.float32)
    total = sum_gt + rem * sum_eq / jnp.maximum(cnt_eq, 1).astype(jnp.float32)
    out_ref[...] = jnp.broadcast_to(total / jnp.float32(_K), (1, 1))


def kernel(y_hat, y):
    nll, tl = _sc_phase1(y_hat.reshape(-1), y.reshape(-1))
    nll2 = nll.reshape(128, 128)
    tl2 = tl.reshape(128, 128)
    out = pl.pallas_call(
        _phase2_body,
        out_shape=jax.ShapeDtypeStruct((1, 1), jnp.float32),
    )(nll2, tl2)
    return out[0, 0]


# speedup vs baseline: 1.2905x; 1.2905x over previous
import functools

import jax
import jax.numpy as jnp
from jax import lax
from jax.experimental import pallas as pl
from jax.experimental.pallas import tpu as pltpu
from jax.experimental.pallas import tpu_sc as plsc

_RATIO = 0.7
_LMBDA = 0.5
_B = 16384
_C = 1000
_K = int(_B * _RATIO)

_NW = 32
_RPW = _B // _NW
_G = 16
_NG = _RPW // _G
_U = 4
_SPAN = _C // _U


def _sc_phase1_body(yh_hbm, y_hbm, nll_hbm, tl_hbm,
                    byh0, byy0, byh1, byy1, snll, stl,
                    semh0, semy0, semh1, semy1):
    wid = lax.axis_index("s") * 2 + lax.axis_index("c")
    base_row = wid * _RPW
    lane = lax.iota(jnp.int32, 16)
    lane_base = lane * _C
    bufs = ((byh0, byy0, semh0, semy0), (byh1, byy1, semh1, semy1))

    def copies(g, b):
        row0 = base_row + g * _G
        byh, byy, semh, semy = bufs[b]
        src_h = yh_hbm.at[pl.ds(row0 * _C, _G * _C)]
        src_y = y_hbm.at[pl.ds(row0 * _C, _G * _C)]
        return (pltpu.make_async_copy(src_h, byh, semh),
                pltpu.make_async_copy(src_y, byy, semy))

    def start(g, b):
        ch, cy = copies(g, b)
        ch.start()
        cy.start()

    def wait(g, b):
        ch, cy = copies(g, b)
        ch.wait()
        cy.wait()

    def compute(g, b):
        byh, byy, _, _ = bufs[b]

        def col_body(j, carry):
            cols, dots, yms, yhms, cbs = carry
            ncols, ndots, nyms, nyhms, ncbs = [], [], [], [], []
            for u in range(_U):
                col = cols[u]
                idx = lane_base + col
                yh = plsc.load_gather(byh, [idx])
                yy = plsc.load_gather(byy, [idx])
                better = (yy > yms[u]) | ((yy == yms[u]) & (col < cbs[u]))
                nyms.append(jnp.where(better, yy, yms[u]))
                nyhms.append(jnp.where(better, yh, yhms[u]))
                ncbs.append(jnp.where(better, col, cbs[u]))
                ndots.append(dots[u] + yh * yy)
                col = col + 1
                ncols.append(jnp.where(col == _C, 0, col))
            return (tuple(ncols), tuple(ndots), tuple(nyms), tuple(nyhms),
                    tuple(ncbs))

        cols0 = tuple(lane + u * _SPAN for u in range(_U))
        zeros = jnp.zeros((16,), jnp.float32)
        init = (cols0,
                (zeros,) * _U,
                (jnp.full((16,), -1.0, jnp.float32),) * _U,
                (zeros,) * _U,
                (jnp.full((16,), _C, jnp.int32),) * _U)
        _, dots, yms, yhms, cbs = lax.fori_loop(0, _SPAN, col_body, init)

        ym, yhm, cb = yms[0], yhms[0], cbs[0]
        dot = dots[0]
        for u in range(1, _U):
            keep = (ym > yms[u]) | ((ym == yms[u]) & (cb < cbs[u]))
            ym = jnp.where(keep, ym, yms[u])
            yhm = jnp.where(keep, yhm, yhms[u])
            cb = jnp.where(keep, cb, cbs[u])
            dot = dot + dots[u]
        nll = -yhm
        snll[pl.ds(g * _G, _G)] = nll
        stl[pl.ds(g * _G, _G)] = nll + _LMBDA * (1.0 - dot)

    start(0, 0)

    def outer(g2, _):
        for b in (0, 1):
            g = 2 * g2 + b

            @pl.when(g + 1 < _NG)
            def _():
                start(g + 1, 1 - b)

            wait(g, b)
            compute(g, b)
        return 0

    lax.fori_loop(0, _NG // 2, outer, 0)
    pltpu.sync_copy(snll, nll_hbm.at[pl.ds(base_row, _RPW)])
    pltpu.sync_copy(stl, tl_hbm.at[pl.ds(base_row, _RPW)])


_sc_phase1 = functools.partial(
    pl.kernel,
    out_type=[
        jax.ShapeDtypeStruct((_B,), jnp.float32),
        jax.ShapeDtypeStruct((_B,), jnp.float32),
    ],
    mesh=plsc.VectorSubcoreMesh(core_axis_name="c", subcore_axis_name="s"),
    scratch_types=[
        pltpu.VMEM((_G * _C,), jnp.float32),
        pltpu.VMEM((_G * _C,), jnp.float32),
        pltpu.VMEM((_G * _C,), jnp.float32),
        pltpu.VMEM((_G * _C,), jnp.float32),
        pltpu.VMEM((_RPW,), jnp.float32),
        pltpu.VMEM((_RPW,), jnp.float32),
        pltpu.SemaphoreType.DMA,
        pltpu.SemaphoreType.DMA,
        pltpu.SemaphoreType.DMA,
        pltpu.SemaphoreType.DMA,
    ],
    compiler_params=pltpu.CompilerParams(needs_layout_passes=False),
)(_sc_phase1_body)


def _phase2_body(nll_ref, tl_ref, out_ref):
    nll = nll_ref[...]
    tl = tl_ref[...]
    i32 = lax.bitcast_convert_type(tl, jnp.int32)
    keyi = jnp.where(i32 < 0, jnp.bitwise_not(i32),
                     jnp.bitwise_or(i32, jnp.int32(-(2**31))))
    u = lax.bitcast_convert_type(keyi, jnp.uint32)
    t = jnp.uint32(0)
    for b in range(31, -1, -1):
        cand = t | jnp.uint32(1 << b)
        cnt = jnp.sum((u >= cand).astype(jnp.int32))
        t = jnp.where(cnt >= _K, cand, t)
    gt = u > t
    eq = u == t
    cnt_gt = jnp.sum(gt.astype(jnp.int32))
    sum_gt = jnp.sum(jnp.where(gt, nll, 0.0))
    cnt_eq = jnp.sum(eq.astype(jnp.int32))
    sum_eq = jnp.sum(jnp.where(eq, nll, 0.0))
    rem = (_K - cnt_gt).astype(jnp.float32)
    total = sum_gt + rem * sum_eq / jnp.maximum(cnt_eq, 1).astype(jnp.float32)
    out_ref[...] = jnp.broadcast_to(total / jnp.float32(_K), (1, 1))


def kernel(y_hat, y):
    nll, tl = _sc_phase1(y_hat.reshape(-1), y.reshape(-1))
    nll2 = nll.reshape(128, 128)
    tl2 = tl.reshape(128, 128)
    out = pl.pallas_call(
        _phase2_body,
        out_shape=jax.ShapeDtypeStruct((1, 1), jnp.float32),
    )(nll2, tl2)
    return out[0, 0]


# speedup vs baseline: 8.6954x; 6.7380x over previous
import jax
import jax.numpy as jnp
from jax import lax
from jax.experimental import pallas as pl

_RATIO = 0.7
_LMBDA = 0.5
_B = 16384
_C = 1000
_K = int(_B * _RATIO)

_BN = 1024
_NB = _B // _BN


def _phase1_body(yh_ref, y_ref, nll_ref, tl_ref):
    yh = yh_ref[...]
    yy = y_ref[...]
    m = jnp.max(yy, axis=0, keepdims=True)
    ii = lax.broadcasted_iota(jnp.int32, yy.shape, 0)
    idx = jnp.min(jnp.where(yy == m, ii, _C), axis=0, keepdims=True)
    nll = -jnp.sum(jnp.where(ii == idx, yh, 0.0), axis=0, keepdims=True)
    dot = jnp.sum(yh * yy, axis=0, keepdims=True)
    nll_ref[...] = nll
    tl_ref[...] = nll + _LMBDA * (1.0 - dot)


def _phase2_body(nll_ref, tl_ref, out_ref):
    nll = nll_ref[...]
    tl = tl_ref[...]
    i32 = lax.bitcast_convert_type(tl, jnp.int32)
    keyi = jnp.where(i32 < 0, jnp.bitwise_not(i32),
                     jnp.bitwise_or(i32, jnp.int32(-(2**31))))
    u = lax.bitcast_convert_type(keyi, jnp.uint32)
    t = jnp.uint32(0)
    for b in range(31, -1, -1):
        cand = t | jnp.uint32(1 << b)
        cnt = jnp.sum((u >= cand).astype(jnp.int32))
        t = jnp.where(cnt >= _K, cand, t)
    gt = u > t
    eq = u == t
    cnt_gt = jnp.sum(gt.astype(jnp.int32))
    sum_gt = jnp.sum(jnp.where(gt, nll, 0.0))
    cnt_eq = jnp.sum(eq.astype(jnp.int32))
    sum_eq = jnp.sum(jnp.where(eq, nll, 0.0))
    rem = (_K - cnt_gt).astype(jnp.float32)
    total = sum_gt + rem * sum_eq / jnp.maximum(cnt_eq, 1).astype(jnp.float32)
    out_ref[...] = jnp.broadcast_to(total / jnp.float32(_K), (1, 1))


def kernel(y_hat, y):
    yht = y_hat.T
    yt = y.T
    nll, tl = pl.pallas_call(
        _phase1_body,
        grid=(_NB,),
        in_specs=[
            pl.BlockSpec((_C, _BN), lambda i: (0, i)),
            pl.BlockSpec((_C, _BN), lambda i: (0, i)),
        ],
        out_specs=[
            pl.BlockSpec((1, _BN), lambda i: (0, i)),
            pl.BlockSpec((1, _BN), lambda i: (0, i)),
        ],
        out_shape=[
            jax.ShapeDtypeStruct((1, _B), jnp.float32),
            jax.ShapeDtypeStruct((1, _B), jnp.float32),
        ],
    )(yht, yt)

    nll2 = nll.reshape(128, 128)
    tl2 = tl.reshape(128, 128)
    out = pl.pallas_call(
        _phase2_body,
        out_shape=jax.ShapeDtypeStruct((1, 1), jnp.float32),
    )(nll2, tl2)
    return out[0, 0]
